# Initial kernel scaffold; baseline (speedup 1.0000x reference)
import functools

import jax
import jax.numpy as jnp
from jax import lax
from jax.experimental import pallas as pl
from jax.experimental.pallas import tpu as pltpu
from jax.experimental.pallas import tpu_sc as plsc

BATCH = 16384
HIST = 50
DIM = 32
B_TOTAL = BATCH * HIST

NUM_WORKERS = 32
PER_W = B_TOTAL // NUM_WORKERS
CHUNK = 1024
N_CHUNKS = PER_W // CHUNK

_mesh = plsc.VectorSubcoreMesh(core_axis_name="c", subcore_axis_name="s")


@functools.partial(
    pl.kernel,
    mesh=_mesh,
    out_type=jax.ShapeDtypeStruct((B_TOTAL, DIM), jnp.float32),
    scratch_types=[
        pltpu.VMEM((CHUNK,), jnp.int32),
        pltpu.VMEM((CHUNK, DIM), jnp.float32),
        pltpu.SemaphoreType.DMA,
    ],
)
def _gather_kernel(table_hbm, ids_hbm, out_hbm, idx_v, rows_v, sem):
    wid = lax.axis_index("s") * 2 + lax.axis_index("c")
    base = wid * PER_W

    def step(i, carry):
        off = base + i * CHUNK
        pltpu.sync_copy(ids_hbm.at[pl.ds(off, CHUNK)], idx_v)
        pltpu.async_copy(table_hbm.at[idx_v], rows_v, sem).wait()
        pltpu.sync_copy(rows_v, out_hbm.at[pl.ds(off, CHUNK)])
        return carry

    lax.fori_loop(0, N_CHUNKS, step, 0)


def kernel(token_ids, embeddings):
    flat = token_ids.reshape(B_TOTAL).astype(jnp.int32)
    out = _gather_kernel(embeddings, flat)
    return out.reshape(BATCH, HIST, DIM)


# speedup vs baseline: 1.0946x; 1.0946x over previous
import functools

import jax
import jax.numpy as jnp
from jax import lax
from jax.experimental import pallas as pl
from jax.experimental.pallas import tpu as pltpu
from jax.experimental.pallas import tpu_sc as plsc

BATCH = 16384
HIST = 50
DIM = 32
B_TOTAL = BATCH * HIST

NUM_WORKERS = 32
PER_W = B_TOTAL // NUM_WORKERS
CHUNK = 1024
N_CHUNKS = PER_W // CHUNK

_mesh = plsc.VectorSubcoreMesh(core_axis_name="c", subcore_axis_name="s")


@functools.partial(
    pl.kernel,
    mesh=_mesh,
    compiler_params=pltpu.CompilerParams(use_tc_tiling_on_sc=False),
    out_type=jax.ShapeDtypeStruct((B_TOTAL, DIM), jnp.float32),
    scratch_types=[
        pltpu.VMEM((CHUNK,), jnp.int32),
        pltpu.VMEM((CHUNK, DIM), jnp.float32),
        pltpu.SemaphoreType.DMA,
    ],
)
def _gather_kernel(table_hbm, ids_hbm, out_hbm, idx_v, rows_v, sem):
    wid = lax.axis_index("s") * 2 + lax.axis_index("c")
    base = wid * PER_W

    def step(i, carry):
        off = base + i * CHUNK
        pltpu.sync_copy(ids_hbm.at[pl.ds(off, CHUNK)], idx_v)
        pltpu.async_copy(table_hbm.at[idx_v], rows_v, sem).wait()
        pltpu.sync_copy(rows_v, out_hbm.at[pl.ds(off, CHUNK)])
        return carry

    lax.fori_loop(0, N_CHUNKS, step, 0)


def kernel(token_ids, embeddings):
    flat = token_ids.reshape(B_TOTAL).astype(jnp.int32)
    out = _gather_kernel(embeddings, flat)
    return out.reshape(BATCH, HIST, DIM)


# speedup vs baseline: 1.1092x; 1.0134x over previous
import functools

import jax
import jax.numpy as jnp
from jax import lax
from jax.experimental import pallas as pl
from jax.experimental.pallas import tpu as pltpu
from jax.experimental.pallas import tpu_sc as plsc

BATCH = 16384
HIST = 50
DIM = 32
B_TOTAL = BATCH * HIST

NUM_WORKERS = 32
PER_W = B_TOTAL // NUM_WORKERS
CHUNK = 1600
N_CHUNKS = PER_W // CHUNK

_mesh = plsc.VectorSubcoreMesh(core_axis_name="c", subcore_axis_name="s")


@functools.partial(
    pl.kernel,
    mesh=_mesh,
    compiler_params=pltpu.CompilerParams(use_tc_tiling_on_sc=False),
    out_type=jax.ShapeDtypeStruct((B_TOTAL, DIM), jnp.float32),
    scratch_types=[
        pltpu.VMEM((2, CHUNK), jnp.int32),
        pltpu.VMEM((2, CHUNK, DIM), jnp.float32),
        pltpu.SemaphoreType.DMA,
        pltpu.SemaphoreType.DMA,
        pltpu.SemaphoreType.DMA,
        pltpu.SemaphoreType.DMA,
    ],
)
def _gather_kernel(table_hbm, ids_hbm, out_hbm, idx_v, rows_v, g0, g1, s0, s1):
    wid = lax.axis_index("s") * 2 + lax.axis_index("c")
    base = wid * PER_W
    sem_g = (g0, g1)
    sem_s = (s0, s1)

    gathers = [None] * N_CHUNKS
    stores = [None] * N_CHUNKS
    for i in range(N_CHUNKS):
        b = i % 2
        off = base + i * CHUNK
        if i >= 2:
            stores[i - 2].wait()
        pltpu.sync_copy(ids_hbm.at[pl.ds(off, CHUNK)], idx_v.at[b])
        gathers[i] = pltpu.async_copy(
            table_hbm.at[idx_v.at[b]], rows_v.at[b], sem_g[b]
        )
        if i >= 1:
            pb = (i - 1) % 2
            poff = base + (i - 1) * CHUNK
            gathers[i - 1].wait()
            stores[i - 1] = pltpu.async_copy(
                rows_v.at[pb], out_hbm.at[pl.ds(poff, CHUNK)], sem_s[pb]
            )
    last = N_CHUNKS - 1
    lb = last % 2
    gathers[last].wait()
    stores[last] = pltpu.async_copy(
        rows_v.at[lb], out_hbm.at[pl.ds(base + last * CHUNK, CHUNK)], sem_s[lb]
    )
    stores[last - 1].wait()
    stores[last].wait()


def kernel(token_ids, embeddings):
    flat = token_ids.reshape(B_TOTAL).astype(jnp.int32)
    out = _gather_kernel(embeddings, flat)
    return out.reshape(BATCH, HIST, DIM)


# speedup vs baseline: 1.9305x; 1.7405x over previous
import functools

import jax
import jax.numpy as jnp
from jax import lax
from jax.experimental import pallas as pl
from jax.experimental.pallas import tpu as pltpu
from jax.experimental.pallas import tpu_sc as plsc

BATCH = 16384
HIST = 50
DIM = 32
B_TOTAL = BATCH * HIST

NUM_WORKERS = 32
PER_W = B_TOTAL // NUM_WORKERS
CHUNK = 1600
N_CHUNKS = PER_W // CHUNK

_mesh = plsc.VectorSubcoreMesh(core_axis_name="c", subcore_axis_name="s")


@functools.partial(
    pl.kernel,
    mesh=_mesh,
    compiler_params=pltpu.CompilerParams(use_tc_tiling_on_sc=False),
    out_type=jax.ShapeDtypeStruct((B_TOTAL, DIM), jnp.float32),
    scratch_types=[
        pltpu.VMEM((2, CHUNK), jnp.int32),
        pltpu.VMEM((2, CHUNK, DIM), jnp.float32),
        pltpu.SemaphoreType.DMA,
        pltpu.SemaphoreType.DMA,
        pltpu.SemaphoreType.DMA,
        pltpu.SemaphoreType.DMA,
    ],
)
def _gather_kernel(table_hbm, ids_hbm, out_hbm, idx_v, rows_v, g0, g1, s0, s1):
    wid = lax.axis_index("s") * 2 + lax.axis_index("c")
    base = wid * PER_W
    sem_g = (g0, g1)
    sem_s = (s0, s1)

    gathers = [None] * N_CHUNKS
    stores = [None] * N_CHUNKS
    for i in range(N_CHUNKS):
        b = i % 2
        off = base + i * CHUNK
        if i >= 2:
            stores[i - 2].wait()
        pltpu.sync_copy(ids_hbm.at[pl.ds(off, CHUNK)], idx_v.at[b])
        gathers[i] = pltpu.async_copy(
            table_hbm.at[idx_v.at[b]], rows_v.at[b], sem_g[b]
        )
        if i >= 1:
            pb = (i - 1) % 2
            poff = base + (i - 1) * CHUNK
            gathers[i - 1].wait()
            stores[i - 1] = pltpu.async_copy(
                rows_v.at[pb], out_hbm.at[pl.ds(poff, CHUNK)], sem_s[pb]
            )
    last = N_CHUNKS - 1
    lb = last % 2
    gathers[last].wait()
    stores[last] = pltpu.async_copy(
        rows_v.at[lb], out_hbm.at[pl.ds(base + last * CHUNK, CHUNK)], sem_s[lb]
    )
    stores[last - 1].wait()
    stores[last].wait()


def kernel(token_ids, embeddings):
    flat = token_ids.T.reshape(B_TOTAL).astype(jnp.int32)
    out = _gather_kernel(embeddings, flat)
    return out.reshape(HIST, BATCH, DIM).transpose(1, 0, 2)
